# baseline (speedup 1.0000x reference)
import functools

import jax
import jax.numpy as jnp
from jax import lax
from jax.experimental import pallas as pl
from jax.experimental.pallas import tpu as pltpu
from jax.experimental.pallas import tpu_sc as plsc

_NUM_CORES = 2
_NUM_SUBCORES = 16
_LANES = 16
_NUM_WORKERS = _NUM_CORES * _NUM_SUBCORES
_CHUNK = 128
_NBUF = 5


def _sc_embed(piece_table, ids_flat, pos_table, type_table, batch, seq, width):
    seq_per_w = seq // _NUM_WORKERS
    chunks = seq_per_w // _CHUNK
    n_items = batch * chunks
    n_type = type_table.shape[0]
    mesh = plsc.VectorSubcoreMesh(core_axis_name="c", subcore_axis_name="s")

    @functools.partial(
        pl.kernel,
        out_type=jax.ShapeDtypeStruct((batch * seq, width), jnp.float32),
        mesh=mesh,
        scratch_types=[
            pltpu.VMEM((n_items, _CHUNK), jnp.int32),
            pltpu.VMEM((seq_per_w, width), jnp.float32),
            pltpu.VMEM((n_type, width), jnp.float32),
        ]
        + [pltpu.VMEM((_CHUNK, width), jnp.float32) for _ in range(_NBUF)]
        + [pltpu.SemaphoreType.DMA for _ in range(2 * _NBUF + 3)],
    )
    def k(tbl_hbm, ids_hbm, pos_hbm, type_hbm, out_hbm, idx_v, pos_v, type_v, *rest):
        rows = rest[:_NBUF]
        g_sem = rest[_NBUF : 2 * _NBUF]
        s_sem = rest[2 * _NBUF : 3 * _NBUF]
        p_sem = rest[3 * _NBUF]
        t_sem = rest[3 * _NBUF + 1]
        i_sem = rest[3 * _NBUF + 2]
        wid = lax.axis_index("s") * _NUM_CORES + lax.axis_index("c")
        seq_base = wid * seq_per_w

        def flat_start(k_):
            b, c = divmod(k_, chunks)
            return b * seq + seq_base + c * _CHUNK

        idx_cps = [
            pltpu.async_copy(
                ids_hbm.at[pl.ds(flat_start(it), _CHUNK)],
                idx_v.at[it],
                i_sem,
            )
            for it in range(n_items)
        ]
        pos_cp = pltpu.async_copy(
            pos_hbm.at[pl.ds(seq_base, seq_per_w)], pos_v, p_sem
        )
        type_cp = pltpu.async_copy(type_hbm, type_v, t_sem)
        for cp in idx_cps:
            cp.wait()
        gathers = [
            pltpu.async_copy(tbl_hbm.at[idx_v.at[it]], rows[it], g_sem[it])
            for it in range(_NBUF - 1)
        ]
        pos_cp.wait()
        type_cp.wait()

        @pl.loop(0, seq_per_w)
        def _pre_loop(r):
            for j in range(0, width, _LANES):
                dst = (pl.ds(r, 1), pl.ds(j, _LANES))
                tsrc = (pl.ds(0, 1), pl.ds(j, _LANES))
                pos_v.at[*dst][...] = (
                    pos_v.at[*dst][...] + type_v.at[*tsrc][...]
                )

        stores = [None] * _NBUF
        for it in range(n_items):
            buf = it % _NBUF
            gathers[it].wait()
            co = (it % chunks) * _CHUNK

            @pl.loop(0, _CHUNK)
            def _row_loop(r):
                for j in range(0, width, _LANES):
                    dst = (pl.ds(r, 1), pl.ds(j, _LANES))
                    src = (pl.ds(co + r, 1), pl.ds(j, _LANES))
                    rows[buf].at[*dst][...] = (
                        rows[buf].at[*dst][...] + pos_v.at[*src][...]
                    )

            stores[buf] = pltpu.async_copy(
                rows[buf], out_hbm.at[pl.ds(flat_start(it), _CHUNK)], s_sem[buf]
            )
            nxt = it + _NBUF - 1
            if nxt < n_items:
                jb = nxt % _NBUF
                if stores[jb] is not None:
                    stores[jb].wait()
                    stores[jb] = None
                gathers.append(
                    pltpu.async_copy(
                        tbl_hbm.at[idx_v.at[nxt]], rows[jb], g_sem[jb]
                    )
                )
        for buf in range(_NBUF):
            if stores[buf] is not None:
                stores[buf].wait()

    return k(piece_table, ids_flat, pos_table, type_table)


def kernel(piece_ids, piece_table, type_table, pos_table):
    batch, seq = piece_ids.shape
    width = piece_table.shape[1]
    ids_flat = piece_ids.reshape(batch * seq)
    out = _sc_embed(
        piece_table, ids_flat, pos_table[:seq], type_table, batch, seq, width
    )
    return out.reshape(batch, seq, width)


# speedup vs baseline: 1.1797x; 1.1797x over previous
import functools

import jax
import jax.numpy as jnp
from jax import lax
from jax.experimental import pallas as pl
from jax.experimental.pallas import tpu as pltpu
from jax.experimental.pallas import tpu_sc as plsc

_NUM_CORES = 2
_NUM_SUBCORES = 16
_LANES = 16
_NUM_WORKERS = _NUM_CORES * _NUM_SUBCORES
_CHUNK = 128
_NBUF = 5


def _combine_body(pos_ref, type_ref, out_ref):
    out_ref[...] = pos_ref[...] + type_ref[0, :][None, :]


def _sc_embed(piece_table, ids_flat, comb, batch, seq, width):
    seq_per_w = seq // _NUM_WORKERS
    chunks = seq_per_w // _CHUNK
    n_items = batch * chunks
    mesh = plsc.VectorSubcoreMesh(core_axis_name="c", subcore_axis_name="s")

    @functools.partial(
        pl.kernel,
        out_type=jax.ShapeDtypeStruct((batch * seq, width), jnp.float32),
        mesh=mesh,
        scratch_types=[
            pltpu.VMEM((n_items, _CHUNK), jnp.int32),
            pltpu.VMEM((seq_per_w, width), jnp.float32),
        ]
        + [pltpu.VMEM((_CHUNK, width), jnp.float32) for _ in range(_NBUF)]
        + [pltpu.SemaphoreType.DMA for _ in range(2 * _NBUF + 2)],
    )
    def k(tbl_hbm, ids_hbm, comb_hbm, out_hbm, idx_v, comb_v, *rest):
        rows = rest[:_NBUF]
        g_sem = rest[_NBUF : 2 * _NBUF]
        s_sem = rest[2 * _NBUF : 3 * _NBUF]
        c_sem = rest[3 * _NBUF]
        i_sem = rest[3 * _NBUF + 1]
        wid = lax.axis_index("s") * _NUM_CORES + lax.axis_index("c")
        seq_base = wid * seq_per_w

        def flat_start(k_):
            b, c = divmod(k_, chunks)
            return b * seq + seq_base + c * _CHUNK

        idx_cps = [
            pltpu.async_copy(
                ids_hbm.at[pl.ds(flat_start(it), _CHUNK)],
                idx_v.at[it],
                i_sem,
            )
            for it in range(n_items)
        ]
        comb_cp = pltpu.async_copy(
            comb_hbm.at[pl.ds(seq_base, seq_per_w)], comb_v, c_sem
        )
        for cp in idx_cps:
            cp.wait()
        gathers = [
            pltpu.async_copy(tbl_hbm.at[idx_v.at[it]], rows[it], g_sem[it])
            for it in range(_NBUF - 1)
        ]
        comb_cp.wait()

        stores = [None] * _NBUF
        for it in range(n_items):
            buf = it % _NBUF
            gathers[it].wait()
            co = (it % chunks) * _CHUNK

            @pl.loop(0, _CHUNK)
            def _row_loop(r):
                for j in range(0, width, _LANES):
                    dst = (pl.ds(r, 1), pl.ds(j, _LANES))
                    src = (pl.ds(co + r, 1), pl.ds(j, _LANES))
                    rows[buf].at[*dst][...] = (
                        rows[buf].at[*dst][...] + comb_v.at[*src][...]
                    )

            stores[buf] = pltpu.async_copy(
                rows[buf], out_hbm.at[pl.ds(flat_start(it), _CHUNK)], s_sem[buf]
            )
            nxt = it + _NBUF - 1
            if nxt < n_items:
                jb = nxt % _NBUF
                if stores[jb] is not None:
                    stores[jb].wait()
                    stores[jb] = None
                gathers.append(
                    pltpu.async_copy(
                        tbl_hbm.at[idx_v.at[nxt]], rows[jb], g_sem[jb]
                    )
                )
        for buf in range(_NBUF):
            if stores[buf] is not None:
                stores[buf].wait()

    return k(piece_table, ids_flat, comb)


def kernel(piece_ids, piece_table, type_table, pos_table):
    batch, seq = piece_ids.shape
    width = piece_table.shape[1]
    comb = pl.pallas_call(
        _combine_body,
        out_shape=jax.ShapeDtypeStruct((seq, width), jnp.float32),
    )(pos_table[:seq], type_table)
    ids_flat = piece_ids.reshape(batch * seq)
    out = _sc_embed(piece_table, ids_flat, comb, batch, seq, width)
    return out.reshape(batch, seq, width)
